# baseline (speedup 1.0000x reference)
import math

import jax
import jax.numpy as jnp
from jax.experimental import pallas as pl

N = 32768
M = int(math.ceil((64.0 / 2048.0) * N))
R = 0.2
K = 128
SLOPE = 0.2


def _leaky(h):
    return jax.nn.leaky_relu(h, negative_slope=SLOPE)


def _fps_idx(pos, m):
    n = pos.shape[0]

    def body(i, state):
        mind, idx, last = state
        d = jnp.sum((pos - pos[last]) ** 2, axis=-1)
        mind = jnp.minimum(mind, d)
        nxt = jnp.argmax(mind).astype(jnp.int32)
        idx = idx.at[i].set(nxt)
        return (mind, idx, nxt)

    idx0 = jnp.zeros((m,), jnp.int32)
    state = (jnp.full((n,), jnp.inf, dtype=jnp.float32), idx0, jnp.int32(0))
    _, idx, _ = jax.lax.fori_loop(1, m, body, state)
    return idx


def _radius_edges(pos, q):
    d2 = (jnp.sum(q * q, axis=1)[:, None]
          + jnp.sum(pos * pos, axis=1)[None, :]
          - 2.0 * q @ pos.T)
    d2 = jnp.maximum(d2, 0.0)
    within = d2 <= R * R
    neg, nbr = jax.lax.top_k(jnp.where(within, -d2, -jnp.inf), K)
    valid = jnp.isfinite(neg)
    x_idx = nbr.reshape(-1).astype(jnp.int32)
    y_idx = jnp.repeat(jnp.arange(M, dtype=jnp.int32), K)
    vmask = valid.reshape(-1)
    return x_idx, y_idx, vmask


def _copy_kernel(a_ref, b_ref, oa_ref, ob_ref):
    oa_ref[...] = a_ref[...]
    ob_ref[...] = b_ref[...]


def kernel(x, pos, batch, W1, b1, W2, b2, W3, b3, W4, b4, W5, b5):
    idx = _fps_idx(pos, M)
    q = pos[idx]
    x_idx, y_idx, vmask = _radius_edges(pos, q)
    rel = pos[x_idx] - q[y_idx]
    h = jnp.concatenate([x[x_idx], rel], axis=-1)
    h = _leaky(h @ W1 + b1)
    h = _leaky(h @ W2 + b2)
    h = _leaky(h @ W3 + b3)
    h = jnp.where(vmask[:, None], h, -jnp.inf)
    agg = jax.ops.segment_max(h, y_idx, num_segments=M)
    agg = jnp.where(jnp.isfinite(agg), agg, 0.0)
    z = _leaky(jnp.concatenate([agg, q], axis=-1) @ W4 + b4)
    z = z @ W5 + b5
    mean, logvar = jnp.split(z, 2, axis=-1)
    std = jnp.exp(0.5 * logvar)
    mean, std = pl.pallas_call(
        _copy_kernel,
        out_shape=(jax.ShapeDtypeStruct(mean.shape, mean.dtype),
                   jax.ShapeDtypeStruct(std.shape, std.dtype)),
    )(mean, std)
    return (mean, std, x_idx, y_idx)


# speedup vs baseline: 1.6726x; 1.6726x over previous
import functools
import math

import jax
import jax.numpy as jnp
from jax.experimental import pallas as pl
from jax.experimental.pallas import tpu as pltpu

N = 32768
M = int(math.ceil((64.0 / 2048.0) * N))
R = 0.2
K = 128
SLOPE = 0.2

_ROWS = N // 128


def _leaky(h):
    return jax.nn.leaky_relu(h, negative_slope=SLOPE)


def _fps_body(m, px_ref, py_ref, pz_ref, idx_ref, mind_ref):
    row = jax.lax.broadcasted_iota(jnp.int32, (_ROWS, 128), 0)
    col = jax.lax.broadcasted_iota(jnp.int32, (_ROWS, 128), 1)
    flat = row * 128 + col
    lane = jax.lax.broadcasted_iota(jnp.int32, (1, 128), 1)
    idx_ref[0] = 0
    mind_ref[...] = jnp.full((_ROWS, 128), jnp.inf, jnp.float32)

    def body(i, last):
        r = last // 128
        c = last % 128

        def pick(ref):
            rowv = ref[pl.ds(r, 1), :]
            return jnp.sum(jnp.where(lane == c, rowv, 0.0))

        lx, ly, lz = pick(px_ref), pick(py_ref), pick(pz_ref)
        dx = px_ref[...] - lx
        dy = py_ref[...] - ly
        dz = pz_ref[...] - lz
        d = (dx * dx + dy * dy) + dz * dz
        mind = jnp.minimum(mind_ref[...], d)
        mind_ref[...] = mind
        maxv = jnp.max(mind)
        nxt = jnp.min(jnp.where(mind == maxv, flat, jnp.int32(2**30)))
        idx_ref[i] = nxt
        return nxt

    jax.lax.fori_loop(1, m, body, jnp.int32(0))


def _fps_idx(pos, m):
    posT = pos.T.reshape(3, _ROWS, 128)
    return pl.pallas_call(
        functools.partial(_fps_body, m),
        out_shape=jax.ShapeDtypeStruct((m,), jnp.int32),
        in_specs=[pl.BlockSpec(memory_space=pltpu.VMEM)] * 3,
        out_specs=pl.BlockSpec(memory_space=pltpu.SMEM),
        scratch_shapes=[pltpu.VMEM((_ROWS, 128), jnp.float32)],
    )(posT[0], posT[1], posT[2])


def _radius_edges(pos, q):
    d2 = (jnp.sum(q * q, axis=1)[:, None]
          + jnp.sum(pos * pos, axis=1)[None, :]
          - 2.0 * q @ pos.T)
    d2 = jnp.maximum(d2, 0.0)
    within = d2 <= R * R
    neg, nbr = jax.lax.top_k(jnp.where(within, -d2, -jnp.inf), K)
    valid = jnp.isfinite(neg)
    x_idx = nbr.reshape(-1).astype(jnp.int32)
    y_idx = jnp.repeat(jnp.arange(M, dtype=jnp.int32), K)
    vmask = valid.reshape(-1)
    return x_idx, y_idx, vmask


def _copy_kernel(a_ref, b_ref, oa_ref, ob_ref):
    oa_ref[...] = a_ref[...]
    ob_ref[...] = b_ref[...]


def kernel(x, pos, batch, W1, b1, W2, b2, W3, b3, W4, b4, W5, b5):
    idx = _fps_idx(pos, M)
    q = pos[idx]
    x_idx, y_idx, vmask = _radius_edges(pos, q)
    rel = pos[x_idx] - q[y_idx]
    h = jnp.concatenate([x[x_idx], rel], axis=-1)
    h = _leaky(h @ W1 + b1)
    h = _leaky(h @ W2 + b2)
    h = _leaky(h @ W3 + b3)
    h = jnp.where(vmask[:, None], h, -jnp.inf)
    agg = jax.ops.segment_max(h, y_idx, num_segments=M)
    agg = jnp.where(jnp.isfinite(agg), agg, 0.0)
    z = _leaky(jnp.concatenate([agg, q], axis=-1) @ W4 + b4)
    z = z @ W5 + b5
    mean, logvar = jnp.split(z, 2, axis=-1)
    std = jnp.exp(0.5 * logvar)
    mean, std = pl.pallas_call(
        _copy_kernel,
        out_shape=(jax.ShapeDtypeStruct(mean.shape, mean.dtype),
                   jax.ShapeDtypeStruct(std.shape, std.dtype)),
    )(mean, std)
    return (mean, std, x_idx, y_idx)


# speedup vs baseline: 1.7917x; 1.0712x over previous
import functools
import math

import jax
import jax.numpy as jnp
from jax.experimental import pallas as pl
from jax.experimental.pallas import tpu as pltpu

N = 32768
M = int(math.ceil((64.0 / 2048.0) * N))
R = 0.2
K = 128
SLOPE = 0.2

_ROWS = N // 128


def _leaky(h):
    return jax.nn.leaky_relu(h, negative_slope=SLOPE)


def _fps_body(m, px_ref, py_ref, pz_ref, idx_ref, mind_ref):
    row = jax.lax.broadcasted_iota(jnp.int32, (_ROWS, 128), 0)
    col = jax.lax.broadcasted_iota(jnp.int32, (_ROWS, 128), 1)
    flat = row * 128 + col
    lane = jax.lax.broadcasted_iota(jnp.int32, (1, 128), 1)
    idx_ref[0] = 0
    mind_ref[...] = jnp.full((_ROWS, 128), jnp.inf, jnp.float32)

    def body(i, last):
        r = last // 128
        c = last % 128

        def pick(ref):
            rowv = ref[pl.ds(r, 1), :]
            return jnp.sum(jnp.where(lane == c, rowv, 0.0))

        lx, ly, lz = pick(px_ref), pick(py_ref), pick(pz_ref)
        dx = px_ref[...] - lx
        dy = py_ref[...] - ly
        dz = pz_ref[...] - lz
        d = (dx * dx + dy * dy) + dz * dz
        mind = jnp.minimum(mind_ref[...], d)
        mind_ref[...] = mind
        maxv = jnp.max(mind)
        nxt = jnp.min(jnp.where(mind == maxv, flat, jnp.int32(2**30)))
        idx_ref[i] = nxt
        return nxt

    jax.lax.fori_loop(1, m, body, jnp.int32(0))


def _fps_idx(pos, m):
    posT = pos.T.reshape(3, _ROWS, 128)
    return pl.pallas_call(
        functools.partial(_fps_body, m),
        out_shape=jax.ShapeDtypeStruct((m,), jnp.int32),
        in_specs=[pl.BlockSpec(memory_space=pltpu.VMEM)] * 3,
        out_specs=pl.BlockSpec(memory_space=pltpu.SMEM),
        scratch_shapes=[pltpu.VMEM((_ROWS, 128), jnp.float32)],
    )(posT[0], posT[1], posT[2])


def _radius_edges(pos, q):
    d2 = (jnp.sum(q * q, axis=1)[:, None]
          + jnp.sum(pos * pos, axis=1)[None, :]
          - 2.0 * q @ pos.T)
    d2 = jnp.maximum(d2, 0.0)
    within = d2 <= R * R
    neg, nbr = jax.lax.top_k(jnp.where(within, -d2, -jnp.inf), K)
    valid = jnp.isfinite(neg)
    x_idx = nbr.reshape(-1).astype(jnp.int32)
    y_idx = jnp.repeat(jnp.arange(M, dtype=jnp.int32), K)
    vmask = valid.reshape(-1)
    return x_idx, y_idx, vmask


_TILE_Q = 16
_TILE_E = _TILE_Q * K


def _edge_mlp_body(rel_ref, msk_ref, w1_ref, b1_ref, w2_ref, b2_ref,
                   w3_ref, b3_ref, agg_ref):
    h = jnp.dot(rel_ref[...], w1_ref[...],
                preferred_element_type=jnp.float32) + b1_ref[...]
    h = _leaky(h)
    h = jnp.dot(h, w2_ref[...], preferred_element_type=jnp.float32) + b2_ref[...]
    h = _leaky(h)
    h = jnp.dot(h, w3_ref[...], preferred_element_type=jnp.float32) + b3_ref[...]
    h = _leaky(h)
    h = jnp.where(msk_ref[...] > 0, h, -jnp.inf)
    agg = jnp.max(h.reshape(_TILE_Q, K, 512), axis=1)
    agg_ref[...] = jnp.where(jnp.isfinite(agg), agg, 0.0)


def _edge_mlp(rel, vmask, W1, b1, W2, b2, W3, b3):
    E = rel.shape[0]
    msk = vmask.astype(jnp.float32).reshape(E, 1)
    grid = E // _TILE_E
    return pl.pallas_call(
        _edge_mlp_body,
        grid=(grid,),
        in_specs=[
            pl.BlockSpec((_TILE_E, 3), lambda i: (i, 0)),
            pl.BlockSpec((_TILE_E, 1), lambda i: (i, 0)),
            pl.BlockSpec((3, 64), lambda i: (0, 0)),
            pl.BlockSpec((64,), lambda i: (0,)),
            pl.BlockSpec((64, 128), lambda i: (0, 0)),
            pl.BlockSpec((128,), lambda i: (0,)),
            pl.BlockSpec((128, 512), lambda i: (0, 0)),
            pl.BlockSpec((512,), lambda i: (0,)),
        ],
        out_specs=pl.BlockSpec((_TILE_Q, 512), lambda i: (i, 0)),
        out_shape=jax.ShapeDtypeStruct((M, 512), jnp.float32),
    )(rel, msk, W1, b1, W2, b2, W3, b3)


def _head_mlp_body(agg_ref, q_ref, w4a_ref, w4b_ref, b4_ref, w5_ref, b5_ref,
                   mean_ref, std_ref):
    z = (jnp.dot(agg_ref[...], w4a_ref[...], preferred_element_type=jnp.float32)
         + jnp.dot(q_ref[...], w4b_ref[...], preferred_element_type=jnp.float32)
         + b4_ref[...])
    z = _leaky(z)
    z = jnp.dot(z, w5_ref[...], preferred_element_type=jnp.float32) + b5_ref[...]
    mean_ref[...] = z[:, :512]
    std_ref[...] = jnp.exp(0.5 * z[:, 512:])


def _head_mlp(agg, q, W4, b4, W5, b5):
    return pl.pallas_call(
        _head_mlp_body,
        out_shape=(jax.ShapeDtypeStruct((M, 512), jnp.float32),
                   jax.ShapeDtypeStruct((M, 512), jnp.float32)),
    )(agg, q, W4[:512], W4[512:], b4, W5, b5)


def kernel(x, pos, batch, W1, b1, W2, b2, W3, b3, W4, b4, W5, b5):
    idx = _fps_idx(pos, M)
    q = pos[idx]
    x_idx, y_idx, vmask = _radius_edges(pos, q)
    rel = pos[x_idx] - q[y_idx]
    agg = _edge_mlp(rel, vmask, W1, b1, W2, b2, W3, b3)
    mean, std = _head_mlp(agg, q, W4, b4, W5, b5)
    return (mean, std, x_idx, y_idx)


# speedup vs baseline: 9.1099x; 5.0844x over previous
import functools
import math

import jax
import jax.numpy as jnp
from jax import lax
from jax.experimental import pallas as pl
from jax.experimental.pallas import tpu as pltpu
from jax.experimental.pallas import tpu_sc as plsc

N = 32768
M = int(math.ceil((64.0 / 2048.0) * N))
R = 0.2
K = 128
SLOPE = 0.2

_ROWS = N // 128


def _leaky(h):
    return jax.nn.leaky_relu(h, negative_slope=SLOPE)


def _fps_body(m, px_ref, py_ref, pz_ref, idx_ref, mind_ref):
    row = jax.lax.broadcasted_iota(jnp.int32, (_ROWS, 128), 0)
    col = jax.lax.broadcasted_iota(jnp.int32, (_ROWS, 128), 1)
    flat = row * 128 + col
    lane = jax.lax.broadcasted_iota(jnp.int32, (1, 128), 1)
    idx_ref[0] = 0
    mind_ref[...] = jnp.full((_ROWS, 128), jnp.inf, jnp.float32)

    def body(i, last):
        r = last // 128
        c = last % 128

        def pick(ref):
            rowv = ref[pl.ds(r, 1), :]
            return jnp.sum(jnp.where(lane == c, rowv, 0.0))

        lx, ly, lz = pick(px_ref), pick(py_ref), pick(pz_ref)
        dx = px_ref[...] - lx
        dy = py_ref[...] - ly
        dz = pz_ref[...] - lz
        d = (dx * dx + dy * dy) + dz * dz
        mind = jnp.minimum(mind_ref[...], d)
        mind_ref[...] = mind
        maxv = jnp.max(mind)
        nxt = jnp.min(jnp.where(mind == maxv, flat, jnp.int32(2**30)))
        idx_ref[i] = nxt
        return nxt

    jax.lax.fori_loop(1, m, body, jnp.int32(0))


def _fps_idx(pos, m):
    posT = pos.T.reshape(3, _ROWS, 128)
    return pl.pallas_call(
        functools.partial(_fps_body, m),
        out_shape=jax.ShapeDtypeStruct((m,), jnp.int32),
        in_specs=[pl.BlockSpec(memory_space=pltpu.VMEM)] * 3,
        out_specs=pl.BlockSpec(memory_space=pltpu.SMEM),
        scratch_shapes=[pltpu.VMEM((_ROWS, 128), jnp.float32)],
    )(posT[0], posT[1], posT[2])


_NW = 32
_QPW = M // _NW
_NV = N // 16
_SMAX = 4096


def _topk_sc_body(d2_hbm, xidx_hbm, cnt_hbm,
                  row_v, ckey_v, cidx_v, ckey2_v, cidx2_v, hist_v, pad_v,
                  cnt_v):
    wid = lax.axis_index("s") * 2 + lax.axis_index("c")
    lane = lax.iota(jnp.int32, 16)
    r2 = jnp.float32(R * R)
    r2bits = jnp.full((16,), R * R, jnp.float32).view(jnp.int32)
    one16 = jnp.ones((16,), jnp.int32)
    zero16 = jnp.zeros((16,), jnp.int32)

    def per_query(qi_local, _carry):
        qi = wid * _QPW + qi_local
        pltpu.sync_copy(d2_hbm.at[qi], row_v)

        def comp_body(i, w):
            v = row_v[pl.ds(i * 16, 16)]
            selb = v <= r2
            pref = plsc.cumsum(jnp.where(selb, one16, zero16))
            sel = selb & (w + pref <= _SMAX)
            dst = w + pref - 1
            plsc.store_scatter(ckey_v, [dst], plsc.bitcast(v, jnp.int32),
                               mask=sel)
            plsc.store_scatter(cidx_v, [dst], i * 16 + lane, mask=sel)
            pc = plsc.all_reduce_population_count(sel)
            return w + lax.reduce_max(pc, (0,))

        S = lax.fori_loop(0, _NV, comp_body, jnp.int32(0))
        plsc.store_scatter(ckey_v, [S + lane], r2bits,
                           mask=jnp.ones((16,), jnp.bool_))
        nv = (S + 15) // 16

        for digit in range(6):
            src_k = ckey_v if digit % 2 == 0 else ckey2_v
            src_i = cidx_v if digit % 2 == 0 else cidx2_v
            dst_k = ckey2_v if digit % 2 == 0 else ckey_v
            dst_i = cidx2_v if digit % 2 == 0 else cidx_v
            hist_v[pl.ds(0, 16)] = zero16
            hist_v[pl.ds(16, 16)] = zero16

            def hist_body(i, _c, src_k=src_k, digit=digit):
                k = src_k[pl.ds(i * 16, 16)]
                d = (k >> (5 * digit)) & 31
                occ, is_last = plsc.scan_count(d)
                plsc.addupdate_scatter(hist_v, [d], occ, mask=is_last)
                return _c

            lax.fori_loop(0, nv, hist_body, jnp.int32(0))

            h0 = hist_v[pl.ds(0, 16)]
            c0 = plsc.cumsum(h0)
            hist_v[pl.ds(0, 16)] = c0 - h0
            h1 = hist_v[pl.ds(16, 16)]
            c1 = plsc.cumsum(h1)
            hist_v[pl.ds(16, 16)] = lax.reduce_max(c0, (0,)) + c1 - h1

            def place_body(i, _c, src_k=src_k, src_i=src_i,
                           dst_k=dst_k, dst_i=dst_i, digit=digit):
                k = src_k[pl.ds(i * 16, 16)]
                x = src_i[pl.ds(i * 16, 16)]
                d = (k >> (5 * digit)) & 31
                occ, is_last = plsc.scan_count(d)
                pos = plsc.load_gather(hist_v, [d]) + occ - 1
                plsc.store_scatter(dst_k, [pos], k)
                plsc.store_scatter(dst_i, [pos], x)
                plsc.addupdate_scatter(hist_v, [d], occ, mask=is_last)
                return _c

            lax.fori_loop(0, nv, place_body, jnp.int32(0))

        kt = jnp.minimum(S, K)
        need = K - kt

        def pad_cond(st):
            return (st[0] < need) & (st[1] < _NV)

        def pad_body(st):
            wpad, ii = st
            v = row_v[pl.ds(ii * 16, 16)]
            nw = v > r2
            pref = plsc.cumsum(jnp.where(nw, one16, zero16))
            sel = nw & (wpad + pref <= need)
            plsc.store_scatter(pad_v, [wpad + pref - 1], ii * 16 + lane,
                               mask=sel)
            pc = plsc.all_reduce_population_count(sel)
            return (wpad + lax.reduce_max(pc, (0,)), ii + 1)

        lax.while_loop(pad_cond, pad_body, (jnp.int32(0), jnp.int32(0)))
        for t in range(8):
            vals = pad_v[pl.ds(t * 16, 16)]
            plsc.store_scatter(cidx_v, [kt + t * 16 + lane], vals,
                               mask=(t * 16 + lane) < need)

        plsc.store_scatter(cnt_v, [zero16 + qi_local], zero16 + kt,
                           mask=lane == 0)
        pltpu.sync_copy(cidx_v.at[pl.ds(0, K)], xidx_hbm.at[qi])
        return _carry

    lax.fori_loop(0, _QPW, per_query, jnp.int32(0))
    pltpu.sync_copy(cnt_v, cnt_hbm.at[pl.ds(wid * _QPW, _QPW)])


def _topk_sc(d2):
    mesh = plsc.VectorSubcoreMesh(core_axis_name="c", subcore_axis_name="s")
    return pl.kernel(
        _topk_sc_body,
        mesh=mesh,
        out_type=[jax.ShapeDtypeStruct((M, K), jnp.int32),
                  jax.ShapeDtypeStruct((M,), jnp.int32)],
        scratch_types=[
            pltpu.VMEM((N,), jnp.float32),
            pltpu.VMEM((_SMAX + 32,), jnp.int32),
            pltpu.VMEM((_SMAX + 32,), jnp.int32),
            pltpu.VMEM((_SMAX + 32,), jnp.int32),
            pltpu.VMEM((_SMAX + 32,), jnp.int32),
            pltpu.VMEM((512,), jnp.int32),
            pltpu.VMEM((K + 16,), jnp.int32),
            pltpu.VMEM((_QPW,), jnp.int32),
        ],
        compiler_params=pltpu.CompilerParams(needs_layout_passes=False),
    )(d2)


def _radius_edges(pos, q):
    d2 = (jnp.sum(q * q, axis=1)[:, None]
          + jnp.sum(pos * pos, axis=1)[None, :]
          - 2.0 * q @ pos.T)
    d2 = jnp.maximum(d2, 0.0)
    nbr, cnt = _topk_sc(d2)
    x_idx = nbr.reshape(-1)
    y_idx = jnp.repeat(jnp.arange(M, dtype=jnp.int32), K)
    vmask = (jnp.arange(K, dtype=jnp.int32)[None, :] < cnt[:, None]).reshape(-1)
    return x_idx, y_idx, vmask


_TILE_Q = 16
_TILE_E = _TILE_Q * K


def _edge_mlp_body(rel_ref, msk_ref, w1_ref, b1_ref, w2_ref, b2_ref,
                   w3_ref, b3_ref, agg_ref):
    h = jnp.dot(rel_ref[...], w1_ref[...],
                preferred_element_type=jnp.float32) + b1_ref[...]
    h = _leaky(h)
    h = jnp.dot(h, w2_ref[...], preferred_element_type=jnp.float32) + b2_ref[...]
    h = _leaky(h)
    h = jnp.dot(h, w3_ref[...], preferred_element_type=jnp.float32) + b3_ref[...]
    h = _leaky(h)
    h = jnp.where(msk_ref[...] > 0, h, -jnp.inf)
    agg = jnp.max(h.reshape(_TILE_Q, K, 512), axis=1)
    agg_ref[...] = jnp.where(jnp.isfinite(agg), agg, 0.0)


def _edge_mlp(rel, vmask, W1, b1, W2, b2, W3, b3):
    E = rel.shape[0]
    msk = vmask.astype(jnp.float32).reshape(E, 1)
    grid = E // _TILE_E
    return pl.pallas_call(
        _edge_mlp_body,
        grid=(grid,),
        in_specs=[
            pl.BlockSpec((_TILE_E, 3), lambda i: (i, 0)),
            pl.BlockSpec((_TILE_E, 1), lambda i: (i, 0)),
            pl.BlockSpec((3, 64), lambda i: (0, 0)),
            pl.BlockSpec((64,), lambda i: (0,)),
            pl.BlockSpec((64, 128), lambda i: (0, 0)),
            pl.BlockSpec((128,), lambda i: (0,)),
            pl.BlockSpec((128, 512), lambda i: (0, 0)),
            pl.BlockSpec((512,), lambda i: (0,)),
        ],
        out_specs=pl.BlockSpec((_TILE_Q, 512), lambda i: (i, 0)),
        out_shape=jax.ShapeDtypeStruct((M, 512), jnp.float32),
    )(rel, msk, W1, b1, W2, b2, W3, b3)


def _head_mlp_body(agg_ref, q_ref, w4a_ref, w4b_ref, b4_ref, w5_ref, b5_ref,
                   mean_ref, std_ref):
    z = (jnp.dot(agg_ref[...], w4a_ref[...], preferred_element_type=jnp.float32)
         + jnp.dot(q_ref[...], w4b_ref[...], preferred_element_type=jnp.float32)
         + b4_ref[...])
    z = _leaky(z)
    z = jnp.dot(z, w5_ref[...], preferred_element_type=jnp.float32) + b5_ref[...]
    mean_ref[...] = z[:, :512]
    std_ref[...] = jnp.exp(0.5 * z[:, 512:])


def _head_mlp(agg, q, W4, b4, W5, b5):
    return pl.pallas_call(
        _head_mlp_body,
        out_shape=(jax.ShapeDtypeStruct((M, 512), jnp.float32),
                   jax.ShapeDtypeStruct((M, 512), jnp.float32)),
    )(agg, q, W4[:512], W4[512:], b4, W5, b5)


def kernel(x, pos, batch, W1, b1, W2, b2, W3, b3, W4, b4, W5, b5):
    idx = _fps_idx(pos, M)
    q = pos[idx]
    x_idx, y_idx, vmask = _radius_edges(pos, q)
    rel = pos[x_idx] - q[y_idx]
    agg = _edge_mlp(rel, vmask, W1, b1, W2, b2, W3, b3)
    mean, std = _head_mlp(agg, q, W4, b4, W5, b5)
    return (mean, std, x_idx, y_idx)


# speedup vs baseline: 15.6520x; 1.7181x over previous
import functools
import math

import jax
import jax.numpy as jnp
from jax import lax
from jax.experimental import pallas as pl
from jax.experimental.pallas import tpu as pltpu
from jax.experimental.pallas import tpu_sc as plsc

N = 32768
M = int(math.ceil((64.0 / 2048.0) * N))
R = 0.2
K = 128
SLOPE = 0.2

_ROWS = N // 128


def _leaky(h):
    return jax.nn.leaky_relu(h, negative_slope=SLOPE)


def _fps_body(m, px_ref, py_ref, pz_ref, idx_ref, mind_ref):
    row = jax.lax.broadcasted_iota(jnp.int32, (_ROWS, 128), 0)
    col = jax.lax.broadcasted_iota(jnp.int32, (_ROWS, 128), 1)
    flat = row * 128 + col
    lane = jax.lax.broadcasted_iota(jnp.int32, (1, 128), 1)
    idx_ref[0] = 0
    mind_ref[...] = jnp.full((_ROWS, 128), jnp.inf, jnp.float32)

    def body(i, last):
        r = last // 128
        c = last % 128

        def pick(ref):
            rowv = ref[pl.ds(r, 1), :]
            return jnp.sum(jnp.where(lane == c, rowv, 0.0))

        lx, ly, lz = pick(px_ref), pick(py_ref), pick(pz_ref)
        dx = px_ref[...] - lx
        dy = py_ref[...] - ly
        dz = pz_ref[...] - lz
        d = (dx * dx + dy * dy) + dz * dz
        mind = jnp.minimum(mind_ref[...], d)
        mind_ref[...] = mind
        maxv = jnp.max(mind)
        nxt = jnp.min(jnp.where(mind == maxv, flat, jnp.int32(2**30)))
        idx_ref[i] = nxt
        return nxt

    jax.lax.fori_loop(1, m, body, jnp.int32(0))


def _fps_idx(pos, m):
    posT = pos.T.reshape(3, _ROWS, 128)
    return pl.pallas_call(
        functools.partial(_fps_body, m),
        out_shape=jax.ShapeDtypeStruct((m,), jnp.int32),
        in_specs=[pl.BlockSpec(memory_space=pltpu.VMEM)] * 3,
        out_specs=pl.BlockSpec(memory_space=pltpu.SMEM),
        scratch_shapes=[pltpu.VMEM((_ROWS, 128), jnp.float32)],
    )(posT[0], posT[1], posT[2])


_NW = 32
_QPW = M // _NW
_NV = N // 16
_SMAX = 4096


def _topk_sc_body(d2_hbm, xidx_hbm, cnt_hbm,
                  row_v, cnts_v, ckeyf_v, cidx_v, ckey2_v, cidx2_v,
                  ckey3_v, cidx3_v, hist_v, pad_v, cnt_v):
    wid = lax.axis_index("s") * 2 + lax.axis_index("c")
    lane = lax.iota(jnp.int32, 16)
    lane0 = lane == 0
    r2 = jnp.float32(R * R)
    r2f16 = jnp.full((16,), R * R, jnp.float32)
    r2bits = r2f16.view(jnp.int32)
    one16 = jnp.ones((16,), jnp.int32)
    zero16 = jnp.zeros((16,), jnp.int32)
    _NB = 32
    _BSCALE = jnp.float32(_NB / (R * R))

    def per_query(qi_local, _carry):
        qi = wid * _QPW + qi_local
        pltpu.sync_copy(d2_hbm.at[qi], row_v)

        @plsc.parallel_loop(0, _NV, unroll=4)
        def _p1(i):
            v = row_v[pl.ds(i * 16, 16)]
            pc = plsc.all_reduce_population_count(v <= r2)
            plsc.store_scatter(cnts_v, [zero16 + i], pc, mask=lane0)

        def p2_body(j, carry):
            h = cnts_v[pl.ds(j * 16, 16)]
            c = plsc.cumsum(h)
            cnts_v[pl.ds(j * 16, 16)] = carry + (c - h)
            return carry + lax.reduce_max(c, (0,))

        S = lax.fori_loop(0, _NV // 16, p2_body, jnp.int32(0))
        scap = jnp.minimum(S, _SMAX)

        @plsc.parallel_loop(0, _NV, unroll=2)
        def _p3(i):
            v = row_v[pl.ds(i * 16, 16)]
            selb = v <= r2
            base = plsc.load_gather(cnts_v, [zero16 + i])
            pref = plsc.cumsum(jnp.where(selb, one16, zero16))
            dst = base + pref - 1
            sel = selb & (dst < _SMAX)
            plsc.store_scatter(ckeyf_v, [dst], v, mask=sel)
            plsc.store_scatter(cidx_v, [dst], i * 16 + lane, mask=sel)

        plsc.store_scatter(ckeyf_v, [scap + lane], r2f16,
                           mask=jnp.ones((16,), jnp.bool_))
        nv = (scap + 15) // 16

        hist_v[pl.ds(0, 16)] = zero16
        hist_v[pl.ds(16, 16)] = zero16

        def ch_body(i, _c):
            v = ckeyf_v[pl.ds(i * 16, 16)]
            b = jnp.minimum((v * _BSCALE).astype(jnp.int32), _NB - 1)
            occ, il = plsc.scan_count(b)
            plsc.addupdate_scatter(hist_v, [b], occ, mask=il)
            return _c

        lax.fori_loop(0, nv, ch_body, jnp.int32(0))
        target = jnp.minimum(S, K)
        tv = zero16 + target
        c0 = plsc.cumsum(hist_v[pl.ds(0, 16)])
        c1 = lax.reduce_max(c0, (0,)) + plsc.cumsum(hist_v[pl.ds(16, 16)])
        bstar = (plsc.all_reduce_population_count(c0 < tv)
                 + plsc.all_reduce_population_count(c1 < tv))

        def c2_body(i, w2):
            v = ckeyf_v[pl.ds(i * 16, 16)]
            x = cidx_v[pl.ds(i * 16, 16)]
            b = jnp.minimum((v * _BSCALE).astype(jnp.int32), _NB - 1)
            sel = b <= bstar
            pref = plsc.cumsum(jnp.where(sel, one16, zero16))
            dst = w2 + pref - 1
            plsc.store_scatter(ckey2_v, [dst], plsc.bitcast(v, jnp.int32),
                               mask=sel)
            plsc.store_scatter(cidx2_v, [dst], x, mask=sel)
            pc = plsc.all_reduce_population_count(sel)
            return w2 + lax.reduce_max(pc, (0,))

        S2 = lax.fori_loop(0, nv, c2_body, jnp.int32(0))
        plsc.store_scatter(ckey2_v, [S2 + lane], r2bits,
                           mask=jnp.ones((16,), jnp.bool_))
        nv2 = (S2 + 15) // 16

        for digit in range(6):
            src_k = ckey2_v if digit % 2 == 0 else ckey3_v
            src_i = cidx2_v if digit % 2 == 0 else cidx3_v
            dst_k = ckey3_v if digit % 2 == 0 else ckey2_v
            dst_i = cidx3_v if digit % 2 == 0 else cidx2_v
            hist_v[pl.ds(0, 16)] = zero16
            hist_v[pl.ds(16, 16)] = zero16

            def hist_body(i, _c, src_k=src_k, digit=digit):
                k = src_k[pl.ds(i * 16, 16)]
                d = (k >> (5 * digit)) & 31
                occ, il = plsc.scan_count(d)
                plsc.addupdate_scatter(hist_v, [d], occ, mask=il)
                return _c

            lax.fori_loop(0, nv2, hist_body, jnp.int32(0))

            h0 = hist_v[pl.ds(0, 16)]
            cc0 = plsc.cumsum(h0)
            hist_v[pl.ds(0, 16)] = cc0 - h0
            h1 = hist_v[pl.ds(16, 16)]
            cc1 = plsc.cumsum(h1)
            hist_v[pl.ds(16, 16)] = lax.reduce_max(cc0, (0,)) + cc1 - h1

            def place_body(i, _c, src_k=src_k, src_i=src_i,
                           dst_k=dst_k, dst_i=dst_i, digit=digit):
                k = src_k[pl.ds(i * 16, 16)]
                x = src_i[pl.ds(i * 16, 16)]
                d = (k >> (5 * digit)) & 31
                occ, il = plsc.scan_count(d)
                pos = plsc.load_gather(hist_v, [d]) + occ - 1
                plsc.store_scatter(dst_k, [pos], k)
                plsc.store_scatter(dst_i, [pos], x)
                plsc.addupdate_scatter(hist_v, [d], occ, mask=il)
                return _c

            lax.fori_loop(0, nv2, place_body, jnp.int32(0))

        kt = jnp.minimum(S, K)
        need = K - kt

        def pad_cond(st):
            return (st[0] < need) & (st[1] < _NV)

        def pad_body(st):
            wpad, ii = st
            v = row_v[pl.ds(ii * 16, 16)]
            nw = v > r2
            pref = plsc.cumsum(jnp.where(nw, one16, zero16))
            sel = nw & (wpad + pref <= need)
            plsc.store_scatter(pad_v, [wpad + pref - 1], ii * 16 + lane,
                               mask=sel)
            pc = plsc.all_reduce_population_count(sel)
            return (wpad + lax.reduce_max(pc, (0,)), ii + 1)

        lax.while_loop(pad_cond, pad_body, (jnp.int32(0), jnp.int32(0)))
        for t in range(8):
            vals = pad_v[pl.ds(t * 16, 16)]
            plsc.store_scatter(cidx2_v, [kt + t * 16 + lane], vals,
                               mask=(t * 16 + lane) < need)

        plsc.store_scatter(cnt_v, [zero16 + qi_local], zero16 + kt,
                           mask=lane0)
        pltpu.sync_copy(cidx2_v.at[pl.ds(0, K)], xidx_hbm.at[qi])
        return _carry

    lax.fori_loop(0, _QPW, per_query, jnp.int32(0))
    pltpu.sync_copy(cnt_v, cnt_hbm.at[pl.ds(wid * _QPW, _QPW)])


def _topk_sc(d2):
    mesh = plsc.VectorSubcoreMesh(core_axis_name="c", subcore_axis_name="s")
    return pl.kernel(
        _topk_sc_body,
        mesh=mesh,
        out_type=[jax.ShapeDtypeStruct((M, K), jnp.int32),
                  jax.ShapeDtypeStruct((M,), jnp.int32)],
        scratch_types=[
            pltpu.VMEM((N,), jnp.float32),
            pltpu.VMEM((_NV + 16,), jnp.int32),
            pltpu.VMEM((_SMAX + 32,), jnp.float32),
            pltpu.VMEM((_SMAX + 32,), jnp.int32),
            pltpu.VMEM((_SMAX + 32,), jnp.int32),
            pltpu.VMEM((_SMAX + 32,), jnp.int32),
            pltpu.VMEM((_SMAX + 32,), jnp.int32),
            pltpu.VMEM((_SMAX + 32,), jnp.int32),
            pltpu.VMEM((32,), jnp.int32),
            pltpu.VMEM((K + 16,), jnp.int32),
            pltpu.VMEM((_QPW,), jnp.int32),
        ],
        compiler_params=pltpu.CompilerParams(needs_layout_passes=False),
    )(d2)


def _radius_edges(pos, q):
    d2 = (jnp.sum(q * q, axis=1)[:, None]
          + jnp.sum(pos * pos, axis=1)[None, :]
          - 2.0 * q @ pos.T)
    d2 = jnp.maximum(d2, 0.0)
    nbr, cnt = _topk_sc(d2)
    x_idx = nbr.reshape(-1)
    y_idx = jnp.repeat(jnp.arange(M, dtype=jnp.int32), K)
    vmask = (jnp.arange(K, dtype=jnp.int32)[None, :] < cnt[:, None]).reshape(-1)
    return x_idx, y_idx, vmask


_TILE_Q = 16
_TILE_E = _TILE_Q * K


def _edge_mlp_body(rel_ref, msk_ref, w1_ref, b1_ref, w2_ref, b2_ref,
                   w3_ref, b3_ref, agg_ref):
    h = jnp.dot(rel_ref[...], w1_ref[...],
                preferred_element_type=jnp.float32) + b1_ref[...]
    h = _leaky(h)
    h = jnp.dot(h, w2_ref[...], preferred_element_type=jnp.float32) + b2_ref[...]
    h = _leaky(h)
    h = jnp.dot(h, w3_ref[...], preferred_element_type=jnp.float32) + b3_ref[...]
    h = _leaky(h)
    h = jnp.where(msk_ref[...] > 0, h, -jnp.inf)
    agg = jnp.max(h.reshape(_TILE_Q, K, 512), axis=1)
    agg_ref[...] = jnp.where(jnp.isfinite(agg), agg, 0.0)


def _edge_mlp(rel, vmask, W1, b1, W2, b2, W3, b3):
    E = rel.shape[0]
    msk = vmask.astype(jnp.float32).reshape(E, 1)
    grid = E // _TILE_E
    return pl.pallas_call(
        _edge_mlp_body,
        grid=(grid,),
        in_specs=[
            pl.BlockSpec((_TILE_E, 3), lambda i: (i, 0)),
            pl.BlockSpec((_TILE_E, 1), lambda i: (i, 0)),
            pl.BlockSpec((3, 64), lambda i: (0, 0)),
            pl.BlockSpec((64,), lambda i: (0,)),
            pl.BlockSpec((64, 128), lambda i: (0, 0)),
            pl.BlockSpec((128,), lambda i: (0,)),
            pl.BlockSpec((128, 512), lambda i: (0, 0)),
            pl.BlockSpec((512,), lambda i: (0,)),
        ],
        out_specs=pl.BlockSpec((_TILE_Q, 512), lambda i: (i, 0)),
        out_shape=jax.ShapeDtypeStruct((M, 512), jnp.float32),
    )(rel, msk, W1, b1, W2, b2, W3, b3)


def _head_mlp_body(agg_ref, q_ref, w4a_ref, w4b_ref, b4_ref, w5_ref, b5_ref,
                   mean_ref, std_ref):
    z = (jnp.dot(agg_ref[...], w4a_ref[...], preferred_element_type=jnp.float32)
         + jnp.dot(q_ref[...], w4b_ref[...], preferred_element_type=jnp.float32)
         + b4_ref[...])
    z = _leaky(z)
    z = jnp.dot(z, w5_ref[...], preferred_element_type=jnp.float32) + b5_ref[...]
    mean_ref[...] = z[:, :512]
    std_ref[...] = jnp.exp(0.5 * z[:, 512:])


def _head_mlp(agg, q, W4, b4, W5, b5):
    return pl.pallas_call(
        _head_mlp_body,
        out_shape=(jax.ShapeDtypeStruct((M, 512), jnp.float32),
                   jax.ShapeDtypeStruct((M, 512), jnp.float32)),
    )(agg, q, W4[:512], W4[512:], b4, W5, b5)


def kernel(x, pos, batch, W1, b1, W2, b2, W3, b3, W4, b4, W5, b5):
    idx = _fps_idx(pos, M)
    q = pos[idx]
    x_idx, y_idx, vmask = _radius_edges(pos, q)
    rel = pos[x_idx] - q[y_idx]
    agg = _edge_mlp(rel, vmask, W1, b1, W2, b2, W3, b3)
    mean, std = _head_mlp(agg, q, W4, b4, W5, b5)
    return (mean, std, x_idx, y_idx)
